# Initial kernel scaffold; baseline (speedup 1.0000x reference)
import jax
import jax.numpy as jnp
from jax.experimental import pallas as pl


def kernel(x, job_table, machine_table, seq_table, W_time, b_time, W_proj, b_proj):
    raise NotImplementedError("write your pallas kernel here")


# speedup vs baseline: 2.7826x; 2.7826x over previous
import functools

import jax
import jax.numpy as jnp
from jax import lax
from jax.experimental import pallas as pl
from jax.experimental.pallas import tpu as pltpu
from jax.experimental.pallas import tpu_sc as plsc

B, J, O = 64, 50, 50
D = 128
N = B * J * O
NC, NS = 2, 16
NW = NC * NS
RPW = N // NW
CH = 200
NCH = RPW // CH
NCC = 8


def _prep_body(job_ref, mach_ref, seq_ref, wt_ref, bt_ref, wp_ref, bp_ref,
               a_ref, c_ref, m_ref, v_ref):
    wp = wp_ref[...]
    w1 = wp[0:128, :]
    w2 = wp[128:256, :]
    w3 = wp[256:384, :]
    w4 = wp[384:512, :]
    bt = bt_ref[...]
    bp = bp_ref[...]
    const = jnp.dot(bt, w4, preferred_element_type=jnp.float32) + bp
    a_ref[...] = jnp.dot(job_ref[...], w1, preferred_element_type=jnp.float32) + const
    c_ref[...] = jnp.dot(seq_ref[...], w3, preferred_element_type=jnp.float32)
    m_ref[...] = jnp.dot(mach_ref[...], w2, preferred_element_type=jnp.float32)
    v_ref[...] = jnp.dot(wt_ref[...], w4, preferred_element_type=jnp.float32)


def _sc_body(xt_hbm, xm_hbm, a_hbm, c_hbm, m_hbm, v_hbm, out_hbm,
             xt_v, xm_v, a_v, c_v, m_v, v_v, buf0, buf1, sem0, sem1):
    wid = lax.axis_index("s") * NC + lax.axis_index("c")
    base = wid * RPW

    pltpu.sync_copy(xt_hbm.at[pl.ds(base, RPW)], xt_v.at[pl.ds(0, RPW)])
    pltpu.sync_copy(xm_hbm.at[pl.ds(base, RPW)], xm_v.at[pl.ds(0, RPW)])
    pltpu.sync_copy(a_hbm, a_v)
    pltpu.sync_copy(c_hbm, c_v)
    pltpu.sync_copy(m_hbm, m_v)
    pltpu.sync_copy(v_hbm, v_v)

    vv = [v_v[0, pl.ds(cc * 16, 16)] for cc in range(NCC)]

    bufs = (buf0, buf1)
    sems = (sem0, sem1)
    pending = [None, None]

    for k in range(NCH):
        p = k % 2
        buf = bufs[p]
        if pending[p] is not None:
            pending[p].wait()

        def jj_body(jj, _, buf=buf, k=k):
            j = lax.rem(k * (CH // O) + jj, J)
            a = [a_v[j, pl.ds(cc * 16, 16)] for cc in range(NCC)]
            nl0 = k * CH + jj * O
            r0 = jj * O

            def o_body(o, _, buf=buf, a=a, nl0=nl0, r0=r0):
                nl = nl0 + o
                m = xm_v[pl.ds(nl, 16)][0]
                t = xt_v[pl.ds(nl, 16)][0]
                r = r0 + o
                for cc in range(NCC):
                    s = pl.ds(cc * 16, 16)
                    buf[r, s] = a[cc] + c_v[o, s] + m_v[m, s] + t * vv[cc]
                return 0

            lax.fori_loop(0, O, o_body, 0)
            return 0

        lax.fori_loop(0, CH // O, jj_body, 0)
        pending[p] = pltpu.async_copy(buf, out_hbm.at[pl.ds(base + k * CH, CH)],
                                      sems[p])
    pending[0].wait()
    pending[1].wait()


@jax.jit
def kernel(x, job_table, machine_table, seq_table, W_time, b_time, W_proj, b_proj):
    f32 = jnp.float32
    a_tab, c_tab, m_tab, v_row = pl.pallas_call(
        _prep_body,
        out_shape=(
            jax.ShapeDtypeStruct((100, D), f32),
            jax.ShapeDtypeStruct((100, D), f32),
            jax.ShapeDtypeStruct((100, D), f32),
            jax.ShapeDtypeStruct((1, D), f32),
        ),
    )(job_table, machine_table, seq_table, W_time,
      b_time.reshape(1, D), W_proj, b_proj.reshape(1, D))

    xt = x[..., 0].reshape(N).astype(f32)
    xm = x[..., 1].reshape(N)

    sc_fn = pl.kernel(
        _sc_body,
        mesh=plsc.VectorSubcoreMesh(core_axis_name="c", subcore_axis_name="s"),
        out_type=jax.ShapeDtypeStruct((N, D), f32),
        scratch_types=[
            pltpu.VMEM((RPW + 16,), f32),
            pltpu.VMEM((RPW + 16,), jnp.int32),
            pltpu.VMEM((100, D), f32),
            pltpu.VMEM((100, D), f32),
            pltpu.VMEM((100, D), f32),
            pltpu.VMEM((1, D), f32),
            pltpu.VMEM((CH, D), f32),
            pltpu.VMEM((CH, D), f32),

            pltpu.SemaphoreType.DMA,
            pltpu.SemaphoreType.DMA,
        ],
    )
    out = sc_fn(xt, xm, a_tab, c_tab, m_tab, v_row)
    return out.reshape(B, J * O, D)


# speedup vs baseline: 3.6947x; 1.3278x over previous
import jax
import jax.numpy as jnp
from jax import lax
from jax.experimental import pallas as pl
from jax.experimental.pallas import tpu as pltpu
from jax.experimental.pallas import tpu_sc as plsc

B, J, O = 64, 50, 50
D = 128
N = B * J * O
NC, NS = 2, 16
NW = NC * NS
RPW = N // NW
CH = 200
NCH = RPW // CH
NG = (CH + 15) // 16


def _prep_body(job_ref, mach_ref, seq_ref, wt_ref, bt_ref, wp_ref, bp_ref,
               base_ref, g_ref):
    f32 = jnp.float32
    wp = wp_ref[...]
    w1 = wp[0:128, :]
    w2 = wp[128:256, :]
    w3 = wp[256:384, :]
    w4 = wp[384:512, :]
    const = jnp.dot(bt_ref[...], w4, preferred_element_type=f32) + bp_ref[...]
    a = jnp.dot(job_ref[...][:J], w1, preferred_element_type=f32) + const
    c = jnp.dot(seq_ref[...][:O], w3, preferred_element_type=f32)
    basef = (a[:, None, :] + c[None, :, :]).reshape(J * O, D)
    base_ref[...] = jnp.concatenate([basef, basef], axis=0)
    m2 = jnp.dot(mach_ref[...], w2, preferred_element_type=f32)
    v = jnp.dot(wt_ref[...], w4, preferred_element_type=f32)
    t_col = lax.broadcasted_iota(jnp.int32, (100, 1, 1), 0).astype(f32)
    g3 = t_col * v.reshape(1, 1, D) + m2[None, :, :]
    g_ref[...] = g3.reshape(100 * 100, D)


NPG = (RPW + 15) // 16 + 1
NPAD = NPG * 16
NSL = (NPAD + 127) // 128


def _sc_body(xf_hbm, base_hbm, g_hbm, out_hbm,
             pt_v, pm_v, t_v, m_v, idx_v, buf0, buf1, buf2,
             semp, sb0, sb1, sb2, sg0, sg1, sg2, so0, so1, so2):
    wid = lax.axis_index("s") * NC + lax.axis_index("c")
    row0 = wid * RPW
    xoff = row0 * 2
    xmax = xoff + 2 * RPW - 1

    def p_body(g, _):
        v = lax.iota(jnp.int32, 16) * 2 + (xoff + g * 32)
        pt_v[pl.ds(g * 16, 16)] = jnp.minimum(v, xmax)
        pm_v[pl.ds(g * 16, 16)] = jnp.minimum(v + 1, xmax)
        return 0
    lax.fori_loop(0, NPG, p_body, 0)

    gh = []
    for g in range(NSL):
        ln = min(128, NPAD - g * 128)
        s = pl.ds(g * 128, ln)
        gh.append(pltpu.async_copy(xf_hbm.at[pt_v.at[s]], t_v.at[s], semp))
        gh.append(pltpu.async_copy(xf_hbm.at[pm_v.at[s]], m_v.at[s], semp))

    bufs = (buf0, buf1, buf2)
    semb = (sb0, sb1, sb2)
    semg = (sg0, sg1, sg2)
    semo = (so0, so1, so2)

    baseh = [None] * NCH
    for k in range(3):
        baseh[k] = pltpu.async_copy(base_hbm.at[pl.ds(k * CH, CH)],
                                    bufs[k], semb[k])
    for h in gh:
        h.wait()

    def i_body(g, _):
        s = pl.ds(g * 16, 16)
        idx_v[s] = t_v[s] * 100 + m_v[s]
        return 0
    lax.fori_loop(0, NPG, i_body, 0)

    gath = [None] * NCH
    outh = [None] * NCH
    for k in range(NCH + 2):
        if 3 <= k < NCH:
            p = k % 3
            outh[k - 3].wait()
            baseh[k] = pltpu.async_copy(base_hbm.at[pl.ds(k * CH, CH)],
                                        bufs[p], semb[p])
        if 1 <= k < NCH + 1:
            kk = k - 1
            p = kk % 3
            baseh[kk].wait()
            gath[kk] = (
                pltpu.async_copy(g_hbm.at[idx_v.at[pl.ds(kk * CH, 104)]],
                                 bufs[p].at[pl.ds(0, 104)], semg[p], add=True),
                pltpu.async_copy(g_hbm.at[idx_v.at[pl.ds(kk * CH + 104, 96)]],
                                 bufs[p].at[pl.ds(104, 96)], semg[p], add=True),
            )
        if 2 <= k < NCH + 2:
            kk = k - 2
            p = kk % 3
            gath[kk][0].wait()
            gath[kk][1].wait()
            outh[kk] = pltpu.async_copy(bufs[p],
                                        out_hbm.at[pl.ds(row0 + kk * CH, CH)],
                                        semo[p])
    for kk in range(NCH - 3, NCH):
        outh[kk].wait()


@jax.jit
def kernel(x, job_table, machine_table, seq_table, W_time, b_time, W_proj, b_proj):
    f32 = jnp.float32
    base2, g_tab = pl.pallas_call(
        _prep_body,
        out_shape=(
            jax.ShapeDtypeStruct((2 * J * O, D), f32),
            jax.ShapeDtypeStruct((100 * 100, D), f32),
        ),
    )(job_table, machine_table, seq_table, W_time,
      b_time.reshape(1, D), W_proj, b_proj.reshape(1, D))

    xf = x.reshape(2 * N)

    sc_fn = pl.kernel(
        _sc_body,
        mesh=plsc.VectorSubcoreMesh(core_axis_name="c", subcore_axis_name="s"),
        out_type=jax.ShapeDtypeStruct((N, D), f32),
        scratch_types=(
            [pltpu.VMEM((NPAD,), jnp.int32) for _ in range(5)]
            + [pltpu.VMEM((CH, D), f32) for _ in range(3)]
            + [pltpu.SemaphoreType.DMA for _ in range(10)]
        ),
    )
    out = sc_fn(xf, base2, g_tab)
    return out.reshape(B, J * O, D)


# speedup vs baseline: 4.2456x; 1.1491x over previous
import jax
import jax.numpy as jnp
from jax import lax
from jax.experimental import pallas as pl
from jax.experimental.pallas import tpu as pltpu
from jax.experimental.pallas import tpu_sc as plsc

B, J, O = 64, 50, 50
D = 128
JO = J * O
JOP = JO + 4
N = B * JO
NC, NS = 2, 16
NW = NC * NS
BPW = B // NW
BSTR = 2560
NGB = BSTR // 16
NPAT = BPW * BSTR
NSL = NPAT // 128
CH = 200
NCB = 13


def _prep_body(job_ref, mach_ref, seq_ref, wt_ref, bt_ref, wp_ref, bp_ref,
               base_ref, g_ref):
    f32 = jnp.float32
    wp = wp_ref[...]
    w1 = wp[0:128, :]
    w2 = wp[128:256, :]
    w3 = wp[256:384, :]
    w4 = wp[384:512, :]
    const = jnp.dot(bt_ref[...], w4, preferred_element_type=f32) + bp_ref[...]
    a = jnp.dot(job_ref[...][:J], w1, preferred_element_type=f32) + const
    c = jnp.dot(seq_ref[...][:O], w3, preferred_element_type=f32)
    basef = (a[:, None, :] + c[None, :, :]).reshape(JO, D)
    base_ref[...] = jnp.concatenate(
        [basef, jnp.zeros((JOP - JO, D), f32)], axis=0)
    m2 = jnp.dot(mach_ref[...], w2, preferred_element_type=f32)
    v = jnp.dot(wt_ref[...], w4, preferred_element_type=f32)
    t_col = lax.broadcasted_iota(jnp.int32, (100, 1, 1), 0).astype(f32)
    g3 = t_col * v.reshape(1, 1, D) + m2[None, :, :]
    g_ref[...] = g3.reshape(100 * 100, D)


def _sc_body(xf_hbm, base_hbm, g_hbm, out_hbm,
             pt_v, pm_v, t_v, m_v, idx_v, buf0, buf1, buf2,
             semp, sb0, sb1, sb2, sg0, sg1, sg2, so0, so1, so2):
    cid = lax.axis_index("c")
    sid = lax.axis_index("s")
    wid = sid * NC + cid
    b0 = wid * BPW
    xoff = b0 * JO * 2

    def p_body(g, _):
        b2 = g // NGB
        gg = g - b2 * NGB
        rr = gg * 16 + lax.iota(jnp.int32, 16)
        rc = jnp.minimum(rr, JO - 1)
        el = xoff + b2 * (JO * 2) + rc * 2
        pt_v[pl.ds(g * 16, 16)] = el
        pm_v[pl.ds(g * 16, 16)] = el + 1
        return 0
    lax.fori_loop(0, BPW * NGB, p_body, 0)

    gh = []
    for g in range(NSL):
        s = pl.ds(g * 128, 128)
        gh.append(pltpu.async_copy(xf_hbm.at[pt_v.at[s]], t_v.at[s], semp))
        gh.append(pltpu.async_copy(xf_hbm.at[pm_v.at[s]], m_v.at[s], semp))
    for h in gh:
        h.wait()

    def i_body(g, _):
        s = pl.ds(g * 16, 16)
        idx_v[s] = t_v[s] * 100 + m_v[s]
        return 0
    lax.fori_loop(0, BPW * NGB, i_body, 0)

    bufs = (buf0, buf1, buf2)
    semb = (sb0, sb1, sb2)
    semg = (sg0, sg1, sg2)
    semo = (so0, so1, so2)

    chunks = [(b2, c * CH, CH if c < NCB - 1 else JOP - (NCB - 1) * CH)
              for b2 in range(BPW) for c in range(NCB)]
    ncv = len(chunks)

    baseh = [None] * ncv
    gath = [None] * ncv
    outh = [None] * ncv
    for k in range(ncv + 2):
        if k < ncv:
            p = k % 3
            if k >= 3:
                outh[k - 3].wait()
            b2, r0, ln = chunks[k]
            baseh[k] = pltpu.async_copy(base_hbm.at[pl.ds(r0, ln)],
                                        bufs[p].at[pl.ds(0, ln)], semb[p])
        if 1 <= k < ncv + 1:
            kk = k - 1
            p = kk % 3
            b2, r0, ln = chunks[kk]
            baseh[kk].wait()
            gath[kk] = pltpu.async_copy(
                g_hbm.at[idx_v.at[pl.ds(b2 * BSTR + r0, ln)]],
                bufs[p].at[pl.ds(0, ln)], semg[p], add=True)
        if 2 <= k < ncv + 2:
            kk = k - 2
            p = kk % 3
            b2, r0, ln = chunks[kk]
            gath[kk].wait()
            outh[kk] = pltpu.async_copy(
                bufs[p].at[pl.ds(0, ln)],
                out_hbm.at[b0 + b2, pl.ds(r0, ln)], semo[p])
    for kk in range(ncv - 3, ncv):
        outh[kk].wait()


@jax.jit
def kernel(x, job_table, machine_table, seq_table, W_time, b_time, W_proj, b_proj):
    f32 = jnp.float32
    base_pat, g_tab = pl.pallas_call(
        _prep_body,
        out_shape=(
            jax.ShapeDtypeStruct((JOP, D), f32),
            jax.ShapeDtypeStruct((100 * 100, D), f32),
        ),
    )(job_table, machine_table, seq_table, W_time,
      b_time.reshape(1, D), W_proj, b_proj.reshape(1, D))

    sc_fn = pl.kernel(
        _sc_body,
        mesh=plsc.VectorSubcoreMesh(core_axis_name="c", subcore_axis_name="s"),
        out_type=jax.ShapeDtypeStruct((B, JOP, D), f32),
        scratch_types=(
            [pltpu.VMEM((NPAT,), jnp.int32) for _ in range(5)]
            + [pltpu.VMEM((CH, D), f32) for _ in range(3)]
            + [pltpu.SemaphoreType.DMA for _ in range(10)]
        ),
    )
    out = sc_fn(x.reshape(2 * N), base_pat, g_tab)
    return out[:, :JO, :]


# speedup vs baseline: 7.3773x; 1.7376x over previous
import jax
import jax.numpy as jnp
from jax import lax
from jax.experimental import pallas as pl
from jax.experimental.pallas import tpu as pltpu
from jax.experimental.pallas import tpu_sc as plsc

B, J, O = 64, 50, 50
D = 128
JO = J * O
JOP = JO + 4
N = B * JO
NC, NS = 2, 16
NW = NC * NS
BPW = B // NW
CH = 200
NCB = 13
JOPAD = 2512
WIDX = BPW * JOPAD
TMR = 200


def _prep_body(job_ref, mach_ref, seq_ref, wt_ref, bt_ref, wp_ref, bp_ref,
               base_ref, tm_ref):
    f32 = jnp.float32
    wp = wp_ref[...]
    w1 = wp[0:128, :]
    w2 = wp[128:256, :]
    w3 = wp[256:384, :]
    w4 = wp[384:512, :]
    const = jnp.dot(bt_ref[...], w4, preferred_element_type=f32) + bp_ref[...]
    a = jnp.dot(job_ref[...][:J], w1, preferred_element_type=f32) + const
    c = jnp.dot(seq_ref[...][:O], w3, preferred_element_type=f32)
    basef = (a[:, None, :] + c[None, :, :]).reshape(JO, D)
    base_ref[...] = jnp.concatenate(
        [basef, jnp.zeros((JOP - JO, D), f32)], axis=0)
    m2 = jnp.dot(mach_ref[...], w2, preferred_element_type=f32)
    v = jnp.dot(wt_ref[...], w4, preferred_element_type=f32)
    t_col = lax.broadcasted_iota(jnp.int32, (100, 1), 0).astype(f32)
    tm_ref[...] = jnp.concatenate([t_col * v.reshape(1, D), m2], axis=0)


def _sc_body(idx_hbm, base_hbm, tm_hbm, out_hbm,
             idx_v, buf0, buf1, buf2, sptm, spb,
             sb0, sb1, sb2, sg0, sg1, sg2, so0, so1, so2):
    cid = lax.axis_index("c")
    sid = lax.axis_index("s")
    wid = sid * NC + cid
    b0 = wid * BPW

    pltpu.sync_copy(idx_hbm.at[pl.ds(b0 * JOPAD, WIDX)],
                    idx_v.at[pl.ds(0, WIDX)])
    pltpu.sync_copy(idx_hbm.at[pl.ds(B * JOPAD + b0 * JOPAD, WIDX)],
                    idx_v.at[pl.ds(WIDX, WIDX)])

    @pl.when(sid == 0)
    def _stage():
        pltpu.sync_copy(tm_hbm, sptm)
        pltpu.sync_copy(base_hbm, spb)
    plsc.subcore_barrier()

    bufs = (buf0, buf1, buf2)
    semb = (sb0, sb1, sb2)
    semg = (sg0, sg1, sg2)
    semo = (so0, so1, so2)

    chunks = [(b2, c * CH, CH if c < NCB - 1 else JOP - (NCB - 1) * CH)
              for b2 in range(BPW) for c in range(NCB)]
    ncv = len(chunks)

    baseh = [None] * ncv
    gath = [None] * ncv
    outh = [None] * ncv
    for k in range(ncv + 2):
        if k < ncv:
            p = k % 3
            if k >= 3:
                outh[k - 3].wait()
            b2, r0, ln = chunks[k]
            baseh[k] = pltpu.async_copy(spb.at[pl.ds(r0, ln)],
                                        bufs[p].at[pl.ds(0, ln)], semb[p])
        if 1 <= k < ncv + 1:
            kk = k - 1
            p = kk % 3
            b2, r0, ln = chunks[kk]
            baseh[kk].wait()
            i0t = b2 * JOPAD + r0
            i0m = WIDX + b2 * JOPAD + r0
            gath[kk] = []
            for i0 in (i0t, i0m):
                gath[kk].append(pltpu.async_copy(
                    sptm.at[idx_v.at[pl.ds(i0, 104)]],
                    bufs[p].at[pl.ds(0, 104)], semg[p], add=True))
                if ln > 104:
                    gath[kk].append(pltpu.async_copy(
                        sptm.at[idx_v.at[pl.ds(i0 + 104, ln - 104)]],
                        bufs[p].at[pl.ds(104, ln - 104)], semg[p], add=True))
        if 2 <= k < ncv + 2:
            kk = k - 2
            p = kk % 3
            b2, r0, ln = chunks[kk]
            for h in gath[kk]:
                h.wait()
            outh[kk] = pltpu.async_copy(
                bufs[p].at[pl.ds(0, ln)],
                out_hbm.at[b0 + b2, pl.ds(r0, ln)], semo[p])
    for kk in range(ncv - 3, ncv):
        outh[kk].wait()


@jax.jit
def kernel(x, job_table, machine_table, seq_table, W_time, b_time, W_proj, b_proj):
    f32 = jnp.float32
    base_pat, tm_tab = pl.pallas_call(
        _prep_body,
        out_shape=(
            jax.ShapeDtypeStruct((JOP, D), f32),
            jax.ShapeDtypeStruct((TMR, D), f32),
        ),
    )(job_table, machine_table, seq_table, W_time,
      b_time.reshape(1, D), W_proj, b_proj.reshape(1, D))

    t_idx = jnp.pad(x[..., 0].reshape(B, JO), ((0, 0), (0, JOPAD - JO)))
    m_idx = jnp.pad(x[..., 1].reshape(B, JO) + 100,
                    ((0, 0), (0, JOPAD - JO)))
    idx = jnp.concatenate(
        [t_idx.reshape(B * JOPAD), m_idx.reshape(B * JOPAD)])

    sc_fn = pl.kernel(
        _sc_body,
        mesh=plsc.VectorSubcoreMesh(core_axis_name="c", subcore_axis_name="s"),
        out_type=jax.ShapeDtypeStruct((B, JOP, D), f32),
        scratch_types=(
            [pltpu.VMEM((2 * WIDX,), jnp.int32)]
            + [pltpu.VMEM((CH, D), f32) for _ in range(3)]
            + [pltpu.VMEM_SHARED((TMR, D), f32)]
            + [pltpu.VMEM_SHARED((JOP, D), f32)]
            + [pltpu.SemaphoreType.DMA for _ in range(9)]
        ),
    )
    out = sc_fn(idx, base_pat, tm_tab)
    return out[:, :JO, :]


# speedup vs baseline: 7.5377x; 1.0217x over previous
import jax
import jax.numpy as jnp
from jax import lax
from jax.experimental import pallas as pl
from jax.experimental.pallas import tpu as pltpu
from jax.experimental.pallas import tpu_sc as plsc

B, J, O = 64, 50, 50
D = 128
JO = J * O
JOP = JO + 4
N = B * JO
NC, NS = 2, 16
NW = NC * NS
BPW = B // NW
CH = 256
NCB = 10
JOPAD = 2512
WIDX = BPW * JOPAD
TMR = 200


def _prep_body(job_ref, mach_ref, seq_ref, wt_ref, bt_ref, wp_ref, bp_ref,
               base_ref, tm_ref):
    f32 = jnp.float32
    wp = wp_ref[...]
    w1 = wp[0:128, :]
    w2 = wp[128:256, :]
    w3 = wp[256:384, :]
    w4 = wp[384:512, :]
    const = jnp.dot(bt_ref[...], w4, preferred_element_type=f32) + bp_ref[...]
    a = jnp.dot(job_ref[...][:J], w1, preferred_element_type=f32) + const
    c = jnp.dot(seq_ref[...][:O], w3, preferred_element_type=f32)
    basef = (a[:, None, :] + c[None, :, :]).reshape(JO, D)
    base_ref[...] = jnp.concatenate(
        [basef, jnp.zeros((JOP - JO, D), f32)], axis=0)
    m2 = jnp.dot(mach_ref[...], w2, preferred_element_type=f32)
    v = jnp.dot(wt_ref[...], w4, preferred_element_type=f32)
    t_col = lax.broadcasted_iota(jnp.int32, (100, 1), 0).astype(f32)
    tm_ref[...] = jnp.concatenate([t_col * v.reshape(1, D), m2], axis=0)


def _sc_body(idx_hbm, base_hbm, tm_hbm, out_hbm,
             idx_v, buf0, buf1, buf2, sptm, spb,
             sb0, sb1, sb2, sg0, sg1, sg2, so0, so1, so2):
    cid = lax.axis_index("c")
    sid = lax.axis_index("s")
    wid = sid * NC + cid
    b0 = wid * BPW

    pltpu.sync_copy(idx_hbm.at[pl.ds(b0 * JOPAD, WIDX)],
                    idx_v.at[pl.ds(0, WIDX)])
    pltpu.sync_copy(idx_hbm.at[pl.ds(B * JOPAD + b0 * JOPAD, WIDX)],
                    idx_v.at[pl.ds(WIDX, WIDX)])

    @pl.when(sid == 0)
    def _stage():
        pltpu.sync_copy(tm_hbm, sptm)
        pltpu.sync_copy(base_hbm, spb)
    plsc.subcore_barrier()

    bufs = (buf0, buf1, buf2)
    semb = (sb0, sb1, sb2)
    semg = (sg0, sg1, sg2)
    semo = (so0, so1, so2)

    chunks = [(b2, c * CH, CH if c < NCB - 1 else JOP - (NCB - 1) * CH)
              for b2 in range(BPW) for c in range(NCB)]
    ncv = len(chunks)

    baseh = [None] * ncv
    gath = [None] * ncv
    outh = [None] * ncv
    for k in range(ncv + 2):
        if k < ncv:
            p = k % 3
            if k >= 3:
                outh[k - 3].wait()
            b2, r0, ln = chunks[k]
            baseh[k] = pltpu.async_copy(spb.at[pl.ds(r0, ln)],
                                        bufs[p].at[pl.ds(0, ln)], semb[p])
        if 1 <= k < ncv + 1:
            kk = k - 1
            p = kk % 3
            b2, r0, ln = chunks[kk]
            baseh[kk].wait()
            i0t = b2 * JOPAD + r0
            i0m = WIDX + b2 * JOPAD + r0
            gath[kk] = []
            for i0 in (i0t, i0m):
                gath[kk].append(pltpu.async_copy(
                    sptm.at[idx_v.at[pl.ds(i0, 128)]],
                    bufs[p].at[pl.ds(0, 128)], semg[p], add=True))
                if ln > 128:
                    gath[kk].append(pltpu.async_copy(
                        sptm.at[idx_v.at[pl.ds(i0 + 128, ln - 128)]],
                        bufs[p].at[pl.ds(128, ln - 128)], semg[p], add=True))
        if 2 <= k < ncv + 2:
            kk = k - 2
            p = kk % 3
            b2, r0, ln = chunks[kk]
            for h in gath[kk]:
                h.wait()
            outh[kk] = pltpu.async_copy(
                bufs[p].at[pl.ds(0, ln)],
                out_hbm.at[b0 + b2, pl.ds(r0, ln)], semo[p])
    for kk in range(ncv - 3, ncv):
        outh[kk].wait()


@jax.jit
def kernel(x, job_table, machine_table, seq_table, W_time, b_time, W_proj, b_proj):
    f32 = jnp.float32
    base_pat, tm_tab = pl.pallas_call(
        _prep_body,
        out_shape=(
            jax.ShapeDtypeStruct((JOP, D), f32),
            jax.ShapeDtypeStruct((TMR, D), f32),
        ),
    )(job_table, machine_table, seq_table, W_time,
      b_time.reshape(1, D), W_proj, b_proj.reshape(1, D))

    t_idx = jnp.pad(x[..., 0].reshape(B, JO), ((0, 0), (0, JOPAD - JO)))
    m_idx = jnp.pad(x[..., 1].reshape(B, JO) + 100,
                    ((0, 0), (0, JOPAD - JO)))
    idx = jnp.concatenate(
        [t_idx.reshape(B * JOPAD), m_idx.reshape(B * JOPAD)])

    sc_fn = pl.kernel(
        _sc_body,
        mesh=plsc.VectorSubcoreMesh(core_axis_name="c", subcore_axis_name="s"),
        out_type=jax.ShapeDtypeStruct((B, JOP, D), f32),
        scratch_types=(
            [pltpu.VMEM((2 * WIDX,), jnp.int32)]
            + [pltpu.VMEM((CH, D), f32) for _ in range(3)]
            + [pltpu.VMEM_SHARED((TMR, D), f32)]
            + [pltpu.VMEM_SHARED((JOP, D), f32)]
            + [pltpu.SemaphoreType.DMA for _ in range(9)]
        ),
    )
    out = sc_fn(idx, base_pat, tm_tab)
    return out[:, :JO, :]
